# baseline (speedup 1.0000x reference)
import functools

import jax
import jax.numpy as jnp
from jax import lax
from jax.experimental import pallas as pl
from jax.experimental.pallas import tpu as pltpu
from jax.experimental.pallas import tpu_sc as plsc

VOCAB = 100000
GLOVE_DIM = 300
MODIF_DIM = 100
OUT_DIM = GLOVE_DIM + MODIF_DIM
MINI = 100
BATCH = 16384

NUM_CORES = 2
NUM_SUBCORES = 16
NUM_WORKERS = NUM_CORES * NUM_SUBCORES
B_PER_W = BATCH // NUM_WORKERS
CHUNK = 128
NUM_CHUNKS = B_PER_W // CHUNK

_MESH = plsc.VectorSubcoreMesh(core_axis_name="c", subcore_axis_name="s")
_PARAMS = pltpu.CompilerParams(use_tc_tiling_on_sc=False)


@functools.partial(
    pl.kernel,
    mesh=_MESH,
    compiler_params=_PARAMS,
    out_type=jax.ShapeDtypeStruct((BATCH, MINI), jnp.float32),
    scratch_types=[
        pltpu.VMEM((CHUNK,), jnp.int32),
        pltpu.VMEM((CHUNK,), jnp.int32),
        pltpu.VMEM((CHUNK, MINI), jnp.float32),
        pltpu.SemaphoreType.DMA,
    ],
)
def _gather_concat(gix_hbm, six_hbm, basic3_hbm, modif_hbm, out_hbm,
                   gix_v, six_v, rows_v, sem):
    wid = lax.axis_index("s") * NUM_CORES + lax.axis_index("c")
    for ci in range(NUM_CHUNKS):
        off = wid * B_PER_W + ci * CHUNK
        pltpu.sync_copy(gix_hbm.at[pl.ds(off, CHUNK)], gix_v)
        pltpu.sync_copy(six_hbm.at[pl.ds(off, CHUNK)], six_v)
        pltpu.async_copy(modif_hbm.at[gix_v], rows_v, sem).wait()
        pltpu.sync_copy(rows_v, out_hbm.at[pl.ds(off, CHUNK)])


def kernel(word_ids, basic, modif):
    idx = word_ids.astype(jnp.int32)
    pos = jnp.arange(BATCH, dtype=jnp.int32)
    gix = idx
    six = pos
    out100 = _gather_concat(gix, six, basic.reshape(VOCAB * 3, MINI), modif)
    m = out100.reshape(BATCH, MODIF_DIM)
    b = jnp.take(basic, idx, axis=0)
    return jnp.concatenate([b, m], axis=-1)


# speedup vs baseline: 1.3154x; 1.3154x over previous
import functools

import jax
import jax.numpy as jnp
from jax import lax
from jax.experimental import pallas as pl
from jax.experimental.pallas import tpu as pltpu
from jax.experimental.pallas import tpu_sc as plsc

VOCAB = 100000
GLOVE_DIM = 300
MODIF_DIM = 100
OUT_DIM = GLOVE_DIM + MODIF_DIM
BATCH = 16384

BU = 96
MU = 80
B_UNITS = VOCAB * GLOVE_DIM // BU
M_UNITS = VOCAB * MODIF_DIM // MU

NUM_CORES = 2
NUM_SUBCORES = 16
NUM_WORKERS = NUM_CORES * NUM_SUBCORES
B_PER_W = BATCH // NUM_WORKERS
CHUNK = 32
NUM_CHUNKS = B_PER_W // CHUNK
GROUPS = CHUNK // 16

_MESH = plsc.VectorSubcoreMesh(core_axis_name="c", subcore_axis_name="s")
_PARAMS = pltpu.CompilerParams(use_tc_tiling_on_sc=False,
                               needs_layout_passes=False)


@functools.partial(
    pl.kernel,
    mesh=_MESH,
    compiler_params=_PARAMS,
    out_type=jax.ShapeDtypeStruct((BATCH, OUT_DIM), jnp.float32),
    scratch_types=[
        pltpu.VMEM((CHUNK,), jnp.int32),
        pltpu.VMEM((4 * CHUNK, BU), jnp.float32),
        pltpu.VMEM((2 * CHUNK, MU), jnp.float32),
        pltpu.VMEM((CHUNK, OUT_DIM), jnp.float32),
        pltpu.SemaphoreType.DMA,
    ],
)
def _gather_concat(idx_hbm, b96_hbm, m80_hbm, out_hbm,
                   ids_v, bgi_v, mgi_v, crows_v, sem):
    wid = lax.axis_index("s") * NUM_CORES + lax.axis_index("c")
    base = wid * B_PER_W
    lane = lax.iota(jnp.int32, 16)

    def chunk_body(ci, carry):
        off = base + ci * CHUNK
        pltpu.sync_copy(idx_hbm.at[pl.ds(off, CHUNK)], ids_v)
        copies = []
        for g in range(GROUPS):
            for q in range(4):
                idsq = plsc.load_gather(
                    ids_v,
                    [16 * g + 4 * q + lax.shift_right_arithmetic(lane, 2)])
                u0q = idsq * 3 + lax.shift_right_arithmetic(idsq, 3)
                gq = u0q + jnp.bitwise_and(lane, 3)
                copies.append(pltpu.async_copy(
                    b96_hbm.at[gq],
                    bgi_v.at[pl.ds(64 * g + 16 * q, 16)], sem))
            for q in range(2):
                idsq = plsc.load_gather(
                    ids_v,
                    [16 * g + 8 * q + lax.shift_right_arithmetic(lane, 1)])
                u0q = idsq + lax.shift_right_arithmetic(idsq, 2)
                mq = u0q + jnp.bitwise_and(lane, 1)
                copies.append(pltpu.async_copy(
                    m80_hbm.at[mq],
                    mgi_v.at[pl.ds(32 * g + 16 * q, 16)], sem))
        for cp in copies:
            cp.wait()

        def row_body(l, carry2):
            id16 = plsc.load_gather(ids_v, [lane * 0 + l])
            owl = 12 * jnp.bitwise_and(id16, 7) + lane
            for k in range(19):
                dst = 284 if k == 18 else 16 * k
                s = owl + dst
                j_lo = dst // BU
                hi = s >= BU * (j_lo + 1)
                rows = jnp.where(hi, 4 * l + (j_lo + 1), 4 * l + j_lo)
                r = jnp.where(hi, s - BU * (j_lo + 1), s - BU * j_lo)
                v = plsc.load_gather(bgi_v, [rows, r])
                crows_v[l, pl.ds(dst, 16)] = v
            oml = 20 * jnp.bitwise_and(id16, 3) + lane
            for k in range(7):
                dst = 84 if k == 6 else 16 * k
                s = oml + dst
                j_lo = dst // MU
                if dst + 75 < MU * (j_lo + 1):
                    rows = lane * 0 + (2 * l + j_lo)
                    r = s - MU * j_lo
                else:
                    hi = s >= MU * (j_lo + 1)
                    rows = jnp.where(hi, 2 * l + (j_lo + 1), 2 * l + j_lo)
                    r = jnp.where(hi, s - MU * (j_lo + 1), s - MU * j_lo)
                v = plsc.load_gather(mgi_v, [rows, r])
                crows_v[l, pl.ds(GLOVE_DIM + dst, 16)] = v
            return carry2

        lax.fori_loop(0, CHUNK, row_body, 0)
        pltpu.sync_copy(crows_v, out_hbm.at[pl.ds(off, CHUNK)])
        return carry

    lax.fori_loop(0, NUM_CHUNKS, chunk_body, 0)


def kernel(word_ids, basic, modif):
    idx = word_ids.astype(jnp.int32)
    b96 = basic.reshape(B_UNITS, BU)
    m80 = modif.reshape(M_UNITS, MU)
    return _gather_concat(idx, b96, m80)


# speedup vs baseline: 1.3265x; 1.0084x over previous
import functools

import jax
import jax.numpy as jnp
from jax import lax
from jax.experimental import pallas as pl
from jax.experimental.pallas import tpu as pltpu
from jax.experimental.pallas import tpu_sc as plsc

VOCAB = 100000
GLOVE_DIM = 300
MODIF_DIM = 100
OUT_DIM = GLOVE_DIM + MODIF_DIM
BATCH = 16384

BU = 96
MU = 80
B_UNITS = VOCAB * GLOVE_DIM // BU
M_UNITS = VOCAB * MODIF_DIM // MU

NUM_CORES = 2
NUM_SUBCORES = 16
NUM_WORKERS = NUM_CORES * NUM_SUBCORES
B_PER_W = BATCH // NUM_WORKERS
CHUNK = 64
NUM_CHUNKS = B_PER_W // CHUNK
GROUPS = CHUNK // 16

_MESH = plsc.VectorSubcoreMesh(core_axis_name="c", subcore_axis_name="s")
_PARAMS = pltpu.CompilerParams(use_tc_tiling_on_sc=False,
                               needs_layout_passes=False)


@functools.partial(
    pl.kernel,
    mesh=_MESH,
    compiler_params=_PARAMS,
    out_type=jax.ShapeDtypeStruct((BATCH, OUT_DIM), jnp.float32),
    scratch_types=[
        pltpu.VMEM((CHUNK,), jnp.int32),
        pltpu.VMEM((4 * CHUNK, BU), jnp.float32),
        pltpu.VMEM((2 * CHUNK, MU), jnp.float32),
        pltpu.VMEM((CHUNK, OUT_DIM), jnp.float32),
        pltpu.SemaphoreType.DMA,
    ],
)
def _gather_concat(idx_hbm, b96_hbm, m80_hbm, out_hbm,
                   ids_v, bgi_v, mgi_v, crows_v, sem):
    wid = lax.axis_index("s") * NUM_CORES + lax.axis_index("c")
    base = wid * B_PER_W
    lane = lax.iota(jnp.int32, 16)

    def chunk_body(ci, carry):
        off = base + ci * CHUNK
        pltpu.sync_copy(idx_hbm.at[pl.ds(off, CHUNK)], ids_v)
        copies = []
        for g in range(GROUPS):
            for q in range(4):
                idsq = plsc.load_gather(
                    ids_v,
                    [16 * g + 4 * q + lax.shift_right_arithmetic(lane, 2)])
                u0q = idsq * 3 + lax.shift_right_arithmetic(idsq, 3)
                gq = u0q + jnp.bitwise_and(lane, 3)
                copies.append(pltpu.async_copy(
                    b96_hbm.at[gq],
                    bgi_v.at[pl.ds(64 * g + 16 * q, 16)], sem))
            for q in range(2):
                idsq = plsc.load_gather(
                    ids_v,
                    [16 * g + 8 * q + lax.shift_right_arithmetic(lane, 1)])
                u0q = idsq + lax.shift_right_arithmetic(idsq, 2)
                mq = u0q + jnp.bitwise_and(lane, 1)
                copies.append(pltpu.async_copy(
                    m80_hbm.at[mq],
                    mgi_v.at[pl.ds(32 * g + 16 * q, 16)], sem))
        for cp in copies:
            cp.wait()

        def row_body(l, carry2):
            id16 = plsc.load_gather(ids_v, [lane * 0 + l])
            owl = 12 * jnp.bitwise_and(id16, 7) + lane
            for k in range(19):
                dst = 284 if k == 18 else 16 * k
                s = owl + dst
                j_lo = dst // BU
                hi = s >= BU * (j_lo + 1)
                rows = jnp.where(hi, 4 * l + (j_lo + 1), 4 * l + j_lo)
                r = jnp.where(hi, s - BU * (j_lo + 1), s - BU * j_lo)
                v = plsc.load_gather(bgi_v, [rows, r])
                crows_v[l, pl.ds(dst, 16)] = v
            oml = 20 * jnp.bitwise_and(id16, 3) + lane
            for k in range(7):
                dst = 84 if k == 6 else 16 * k
                s = oml + dst
                j_lo = dst // MU
                if dst + 75 < MU * (j_lo + 1):
                    rows = lane * 0 + (2 * l + j_lo)
                    r = s - MU * j_lo
                else:
                    hi = s >= MU * (j_lo + 1)
                    rows = jnp.where(hi, 2 * l + (j_lo + 1), 2 * l + j_lo)
                    r = jnp.where(hi, s - MU * (j_lo + 1), s - MU * j_lo)
                v = plsc.load_gather(mgi_v, [rows, r])
                crows_v[l, pl.ds(GLOVE_DIM + dst, 16)] = v
            return carry2

        lax.fori_loop(0, CHUNK, row_body, 0)
        pltpu.sync_copy(crows_v, out_hbm.at[pl.ds(off, CHUNK)])
        return carry

    lax.fori_loop(0, NUM_CHUNKS, chunk_body, 0)


def kernel(word_ids, basic, modif):
    idx = word_ids.astype(jnp.int32)
    b96 = basic.reshape(B_UNITS, BU)
    m80 = modif.reshape(M_UNITS, MU)
    return _gather_concat(idx, b96, m80)


# speedup vs baseline: 1.3412x; 1.0110x over previous
import functools

import jax
import jax.numpy as jnp
from jax import lax
from jax.experimental import pallas as pl
from jax.experimental.pallas import tpu as pltpu
from jax.experimental.pallas import tpu_sc as plsc

VOCAB = 100000
GLOVE_DIM = 300
MODIF_DIM = 100
OUT_DIM = GLOVE_DIM + MODIF_DIM
BATCH = 16384

BU = 96
MU = 80
B_UNITS = VOCAB * GLOVE_DIM // BU
M_UNITS = VOCAB * MODIF_DIM // MU

NUM_CORES = 2
NUM_SUBCORES = 16
NUM_WORKERS = NUM_CORES * NUM_SUBCORES
B_PER_W = BATCH // NUM_WORKERS
CHUNK = 64
NUM_CHUNKS = B_PER_W // CHUNK
GROUPS = CHUNK // 16

_MESH = plsc.VectorSubcoreMesh(core_axis_name="c", subcore_axis_name="s")
_PARAMS = pltpu.CompilerParams(use_tc_tiling_on_sc=False,
                               needs_layout_passes=False)


@functools.partial(
    pl.kernel,
    mesh=_MESH,
    compiler_params=_PARAMS,
    out_type=jax.ShapeDtypeStruct((BATCH, OUT_DIM), jnp.float32),
    scratch_types=[
        pltpu.VMEM((2, CHUNK), jnp.int32),
        pltpu.VMEM((2, 4 * CHUNK, BU), jnp.float32),
        pltpu.VMEM((2, 2 * CHUNK, MU), jnp.float32),
        pltpu.VMEM((CHUNK, OUT_DIM), jnp.float32),
        pltpu.SemaphoreType.DMA,
        pltpu.SemaphoreType.DMA,
    ],
)
def _gather_concat(idx_hbm, b96_hbm, m80_hbm, out_hbm,
                   ids_v, bgi_v, mgi_v, crows_v, sem0, sem1):
    wid = lax.axis_index("s") * NUM_CORES + lax.axis_index("c")
    base = wid * B_PER_W
    lane = lax.iota(jnp.int32, 16)
    sems = (sem0, sem1)

    def issue(ci, p):
        off = base + ci * CHUNK
        pltpu.sync_copy(idx_hbm.at[pl.ds(off, CHUNK)], ids_v.at[p])
        copies = []
        for g in range(GROUPS):
            for q in range(4):
                idsq = plsc.load_gather(
                    ids_v.at[p],
                    [16 * g + 4 * q + lax.shift_right_arithmetic(lane, 2)])
                u0q = idsq * 3 + lax.shift_right_arithmetic(idsq, 3)
                gq = u0q + jnp.bitwise_and(lane, 3)
                copies.append(pltpu.async_copy(
                    b96_hbm.at[gq],
                    bgi_v.at[p].at[pl.ds(64 * g + 16 * q, 16)], sems[p]))
            for q in range(2):
                idsq = plsc.load_gather(
                    ids_v.at[p],
                    [16 * g + 8 * q + lax.shift_right_arithmetic(lane, 1)])
                u0q = idsq + lax.shift_right_arithmetic(idsq, 2)
                mq = u0q + jnp.bitwise_and(lane, 1)
                copies.append(pltpu.async_copy(
                    m80_hbm.at[mq],
                    mgi_v.at[p].at[pl.ds(32 * g + 16 * q, 16)], sems[p]))
        return copies

    def extract(ci, p):
        off = base + ci * CHUNK
        bgp = bgi_v.at[p]
        mgp = mgi_v.at[p]

        def row_body(l, carry2):
            id16 = plsc.load_gather(ids_v.at[p], [lane * 0 + l])
            owl = 12 * jnp.bitwise_and(id16, 7) + lane
            for k in range(19):
                dst = 284 if k == 18 else 16 * k
                s = owl + dst
                j_lo = dst // BU
                hi = s >= BU * (j_lo + 1)
                rows = jnp.where(hi, 4 * l + (j_lo + 1), 4 * l + j_lo)
                r = jnp.where(hi, s - BU * (j_lo + 1), s - BU * j_lo)
                crows_v[l, pl.ds(dst, 16)] = plsc.load_gather(bgp, [rows, r])
            oml = 20 * jnp.bitwise_and(id16, 3) + lane
            for k in range(7):
                dst = 84 if k == 6 else 16 * k
                s = oml + dst
                j_lo = dst // MU
                if dst + 75 < MU * (j_lo + 1):
                    rows = lane * 0 + (2 * l + j_lo)
                    r = s - MU * j_lo
                else:
                    hi = s >= MU * (j_lo + 1)
                    rows = jnp.where(hi, 2 * l + (j_lo + 1), 2 * l + j_lo)
                    r = jnp.where(hi, s - MU * (j_lo + 1), s - MU * j_lo)
                crows_v[l, pl.ds(GLOVE_DIM + dst, 16)] = (
                    plsc.load_gather(mgp, [rows, r]))
            return carry2

        lax.fori_loop(0, CHUNK, row_body, 0)
        pltpu.sync_copy(crows_v, out_hbm.at[pl.ds(off, CHUNK)])

    pending = issue(0, 0)
    for ci in range(NUM_CHUNKS):
        p = ci % 2
        if ci + 1 < NUM_CHUNKS:
            nxt = issue(ci + 1, 1 - p)
        for cp in pending:
            cp.wait()
        extract(ci, p)
        if ci + 1 < NUM_CHUNKS:
            pending = nxt


def kernel(word_ids, basic, modif):
    idx = word_ids.astype(jnp.int32)
    b96 = basic.reshape(B_UNITS, BU)
    m80 = modif.reshape(M_UNITS, MU)
    return _gather_concat(idx, b96, m80)
